# baseline (speedup 1.0000x reference)
import functools

import jax
import jax.numpy as jnp
from jax import lax
from jax.experimental import pallas as pl
from jax.experimental.pallas import tpu as pltpu
from jax.experimental.pallas import tpu_sc as plsc

BATCH = 16384
DIM = 512

NC = 2
NS = 16
NW = NC * NS
BPW = BATCH // NW
CHUNK = 64
NCHUNK = BPW // CHUNK
NBUF = 3

_mesh = plsc.VectorSubcoreMesh(
    core_axis_name="c", subcore_axis_name="s", num_cores=NC, num_subcores=NS)


@functools.partial(
    pl.kernel,
    out_type=jax.ShapeDtypeStruct((BATCH, DIM), jnp.float32),
    mesh=_mesh,
    scratch_types=[
        pltpu.VMEM((NCHUNK, CHUNK), jnp.int32),
        pltpu.VMEM((NBUF, CHUNK, DIM), jnp.float32),
        pltpu.SemaphoreType.DMA,
        pltpu.SemaphoreType.DMA,
        pltpu.SemaphoreType.DMA,
        pltpu.SemaphoreType.DMA,
        pltpu.SemaphoreType.DMA,
        pltpu.SemaphoreType.DMA,
    ],
)
def _embed_sc(cond_hbm, w_hbm, out_hbm, idx_v, rows_v,
              gs0, gs1, gs2, ws0, ws1, ws2):
    wid = lax.axis_index("s") * NC + lax.axis_index("c")
    base = wid * BPW
    gsems = [gs0, gs1, gs2]
    wsems = [ws0, ws1, ws2]

    pltpu.sync_copy(cond_hbm.at[wid], idx_v)

    gh = [None] * NBUF
    wh = [None] * NBUF

    def start_gather(ch):
        p = ch % NBUF
        if wh[p] is not None:
            wh[p].wait()
        gh[p] = pltpu.async_copy(w_hbm.at[idx_v.at[ch]], rows_v.at[p],
                                 gsems[p])

    for ch in range(min(NBUF, NCHUNK)):
        start_gather(ch)
    for ch in range(NCHUNK):
        p = ch % NBUF
        gh[p].wait()
        wh[p] = pltpu.async_copy(rows_v.at[p],
                                 out_hbm.at[pl.ds(base + ch * CHUNK, CHUNK)],
                                 wsems[p])
        nxt = ch + NBUF
        if nxt < NCHUNK:
            start_gather(nxt)
    for ch in range(max(0, NCHUNK - NBUF), NCHUNK):
        wh[ch % NBUF].wait()


def kernel(condition, embed_weight):
    cond3 = condition.astype(jnp.int32).reshape(NW, NCHUNK, CHUNK)
    return _embed_sc(cond3, embed_weight)


# speedup vs baseline: 10.5054x; 10.5054x over previous
import functools

import jax
import jax.numpy as jnp
from jax import lax
from jax.experimental import pallas as pl
from jax.experimental.pallas import tpu as pltpu
from jax.experimental.pallas import tpu_sc as plsc

BATCH = 16384
DIM = 512
L = 16

NC = 2
NS = 16
NW = NC * NS
BPW = BATCH // NW
CHUNK = 64
NCHUNK = BPW // CHUNK
NBUF = 3

_mesh = plsc.VectorSubcoreMesh(
    core_axis_name="c", subcore_axis_name="s", num_cores=NC, num_subcores=NS)


@functools.partial(
    pl.kernel,
    out_type=jax.ShapeDtypeStruct((BATCH, DIM), jnp.float32),
    mesh=_mesh,
    scratch_types=[
        pltpu.VMEM((BPW,), jnp.int32),
        pltpu.VMEM((2, DIM), jnp.float32),
        pltpu.VMEM((NBUF, CHUNK, DIM), jnp.float32),
        pltpu.SemaphoreType.DMA,
        pltpu.SemaphoreType.DMA,
        pltpu.SemaphoreType.DMA,
    ],
)
def _embed_sc(cond_hbm, w_hbm, out_hbm, cond_v, wv, rows_v, ws0, ws1, ws2):
    wid = lax.axis_index("s") * NC + lax.axis_index("c")
    base = wid * BPW
    wsems = [ws0, ws1, ws2]

    pltpu.sync_copy(cond_hbm.at[wid], cond_v)
    pltpu.sync_copy(w_hbm, wv)

    wh = [None] * NBUF
    for ch in range(NCHUNK):
        p = ch % NBUF
        if wh[p] is not None:
            wh[p].wait()
        for g in range(CHUNK // L):
            cvec = cond_v[pl.ds(ch * CHUNK + g * L, L)].astype(jnp.float32)
            cf = [
                cvec.at[jnp.full((L,), r, jnp.int32)].get(
                    mode="promise_in_bounds")
                for r in range(L)
            ]
            cg = [1.0 - cf[r] for r in range(L)]

            @plsc.parallel_loop(0, DIM // L, step=1, unroll=2)
            def _col(j, _g=g, _p=p, _cf=cf, _cg=cg):
                w0j = wv[0, pl.ds(j * L, L)]
                w1j = wv[1, pl.ds(j * L, L)]
                for r in range(L):
                    rows_v[_p, _g * L + r, pl.ds(j * L, L)] = (
                        _cf[r] * w1j + _cg[r] * w0j)

        wh[p] = pltpu.async_copy(
            rows_v.at[p], out_hbm.at[pl.ds(base + ch * CHUNK, CHUNK)],
            wsems[p])
    for ch in range(max(0, NCHUNK - NBUF), NCHUNK):
        wh[ch % NBUF].wait()


def kernel(condition, embed_weight):
    cond2 = condition.astype(jnp.int32).reshape(NW, BPW)
    return _embed_sc(cond2, embed_weight)


# speedup vs baseline: 10.5207x; 1.0015x over previous
import functools

import jax
import jax.numpy as jnp
from jax import lax
from jax.experimental import pallas as pl
from jax.experimental.pallas import tpu as pltpu
from jax.experimental.pallas import tpu_sc as plsc

BATCH = 16384
DIM = 512
L = 16

NC = 2
NS = 16
NW = NC * NS
BPW = BATCH // NW
CHUNK = 64
NCHUNK = BPW // CHUNK
NBUF = 3

_mesh = plsc.VectorSubcoreMesh(
    core_axis_name="c", subcore_axis_name="s", num_cores=NC, num_subcores=NS)


@functools.partial(
    pl.kernel,
    out_type=jax.ShapeDtypeStruct((BATCH, DIM), jnp.float32),
    mesh=_mesh,
    scratch_types=[
        pltpu.VMEM((BPW,), jnp.int32),
        pltpu.VMEM((2, DIM), jnp.float32),
        pltpu.VMEM((NBUF, CHUNK, DIM), jnp.float32),
        pltpu.SemaphoreType.DMA,
        pltpu.SemaphoreType.DMA,
        pltpu.SemaphoreType.DMA,
    ],
)
def _embed_sc(cond_hbm, w_hbm, out_hbm, cond_v, wv, rows_v, ws0, ws1, ws2):
    wid = lax.axis_index("s") * NC + lax.axis_index("c")
    base = wid * BPW
    wsems = [ws0, ws1, ws2]

    pltpu.sync_copy(cond_hbm.at[pl.ds(base, BPW)], cond_v)
    pltpu.sync_copy(w_hbm, wv)

    wh = [None] * NBUF
    for ch in range(NCHUNK):
        p = ch % NBUF
        if wh[p] is not None:
            wh[p].wait()
        for g in range(CHUNK // L):
            cvec = cond_v[pl.ds(ch * CHUNK + g * L, L)].astype(jnp.float32)
            cf = [
                cvec.at[jnp.full((L,), r, jnp.int32)].get(
                    mode="promise_in_bounds")
                for r in range(L)
            ]
            cg = [1.0 - cf[r] for r in range(L)]

            @plsc.parallel_loop(0, DIM // L, step=1, unroll=2)
            def _col(j, _g=g, _p=p, _cf=cf, _cg=cg):
                w0j = wv[0, pl.ds(j * L, L)]
                w1j = wv[1, pl.ds(j * L, L)]
                for r in range(L):
                    rows_v[_p, _g * L + r, pl.ds(j * L, L)] = (
                        _cf[r] * w1j + _cg[r] * w0j)

        wh[p] = pltpu.async_copy(
            rows_v.at[p], out_hbm.at[pl.ds(base + ch * CHUNK, CHUNK)],
            wsems[p])
    for ch in range(max(0, NCHUNK - NBUF), NCHUNK):
        wh[ch % NBUF].wait()


def kernel(condition, embed_weight):
    return _embed_sc(condition, embed_weight)
